# Initial kernel scaffold; baseline (speedup 1.0000x reference)
import jax
import jax.numpy as jnp
from jax.experimental import pallas as pl


def kernel(input_variable, table):
    raise NotImplementedError("write your pallas kernel here")


# speedup vs baseline: 1.4583x; 1.4583x over previous
import functools

import jax
import jax.numpy as jnp
from jax import lax
from jax.experimental import pallas as pl
from jax.experimental.pallas import tpu as pltpu
from jax.experimental.pallas import tpu_sc as plsc

_BATCH = 4096
_HIST = 200
_EMBED = 32
_B = _BATCH * _HIST
_NC = 2
_NS = 16
_NW = _NC * _NS
_B_PER_W = _B // _NW
_CHUNK = 1024
_N_CHUNKS = _B_PER_W // _CHUNK


def _build():
    mesh = plsc.VectorSubcoreMesh(core_axis_name="c", subcore_axis_name="s")

    @functools.partial(
        pl.kernel,
        mesh=mesh,
        out_type=jax.ShapeDtypeStruct((_B, _EMBED), jnp.float32),
        scratch_types=[
            pltpu.VMEM((_CHUNK,), jnp.int32),
            pltpu.VMEM((_CHUNK, _EMBED), jnp.float32),
            pltpu.SemaphoreType.DMA,
        ],
        compiler_params=pltpu.CompilerParams(use_tc_tiling_on_sc=False),
    )
    def gather_kernel(idx_hbm, table_hbm, out_hbm, idx_v, rows_v, sem):
        wid = lax.axis_index("s") * _NC + lax.axis_index("c")
        base = wid * _B_PER_W

        def body(i, carry):
            off = base + i * _CHUNK
            pltpu.sync_copy(idx_hbm.at[pl.ds(off, _CHUNK)], idx_v)
            pltpu.async_copy(table_hbm.at[idx_v], rows_v, sem).wait()
            pltpu.sync_copy(rows_v, out_hbm.at[pl.ds(off, _CHUNK)])
            return carry

        lax.fori_loop(0, _N_CHUNKS, body, 0)

    return gather_kernel


_gather = _build()


@jax.jit
def kernel(input_variable, table):
    idx = input_variable.reshape(-1).astype(jnp.int32)
    out = _gather(idx, table)
    return out.reshape(_BATCH, _HIST, _EMBED)


# speedup vs baseline: 1.4924x; 1.0234x over previous
import functools

import jax
import jax.numpy as jnp
from jax import lax
from jax.experimental import pallas as pl
from jax.experimental.pallas import tpu as pltpu
from jax.experimental.pallas import tpu_sc as plsc

_BATCH = 4096
_HIST = 200
_EMBED = 32
_B = _BATCH * _HIST
_NC = 2
_NS = 16
_NW = _NC * _NS
_B_PER_W = _B // _NW
_NBUF = 4
_CHUNK = 800
_N_CHUNKS = _B_PER_W // _CHUNK
_N_GROUPS = _N_CHUNKS // _NBUF


def _build():
    mesh = plsc.VectorSubcoreMesh(core_axis_name="c", subcore_axis_name="s")

    @functools.partial(
        pl.kernel,
        mesh=mesh,
        out_type=jax.ShapeDtypeStruct((_B, _EMBED), jnp.float32),
        scratch_types=[
            pltpu.VMEM((_B_PER_W,), jnp.int32),
            [pltpu.VMEM((_CHUNK, _EMBED), jnp.float32) for _ in range(_NBUF)],
            [pltpu.SemaphoreType.DMA for _ in range(_NBUF)],
            [pltpu.SemaphoreType.DMA for _ in range(_NBUF)],
        ],
        compiler_params=pltpu.CompilerParams(use_tc_tiling_on_sc=False),
    )
    def gather_kernel(idx_hbm, table_hbm, out_hbm, idx_v, rows, gsem, ssem):
        wid = lax.axis_index("s") * _NC + lax.axis_index("c")
        base = wid * _B_PER_W
        pltpu.sync_copy(idx_hbm.at[pl.ds(base, _B_PER_W)], idx_v)

        def start_gather(chunk, b):
            pltpu.make_async_copy(
                table_hbm.at[idx_v.at[pl.ds(chunk * _CHUNK, _CHUNK)]],
                rows[b], gsem[b]).start()

        def wait_gather(b):
            pltpu.make_async_copy(
                table_hbm.at[idx_v.at[pl.ds(0, _CHUNK)]],
                rows[b], gsem[b]).wait()

        def start_store(chunk, b):
            pltpu.make_async_copy(
                rows[b], out_hbm.at[pl.ds(base + chunk * _CHUNK, _CHUNK)],
                ssem[b]).start()

        def wait_store(b):
            pltpu.make_async_copy(
                rows[b], out_hbm.at[pl.ds(base, _CHUNK)], ssem[b]).wait()

        for b in range(_NBUF):
            start_gather(b, b)

        def body(i, carry):
            for b in range(_NBUF):
                wait_gather(b)
                start_store(i * _NBUF + b, b)

            @pl.when(i < _N_GROUPS - 1)
            def _():
                for b in range(_NBUF):
                    wait_store(b)
                    start_gather((i + 1) * _NBUF + b, b)

            return carry

        lax.fori_loop(0, _N_GROUPS, body, 0)
        for b in range(_NBUF):
            wait_store(b)

    return gather_kernel


_gather = _build()


@jax.jit
def kernel(input_variable, table):
    idx = input_variable.reshape(-1).astype(jnp.int32)
    out = _gather(idx, table)
    return out.reshape(_BATCH, _HIST, _EMBED)
